# baseline (speedup 1.0000x reference)
import functools

import jax
import jax.numpy as jnp
from jax import lax
from jax.experimental import pallas as pl
from jax.experimental.pallas import tpu as pltpu
from jax.experimental.pallas import tpu_sc as plsc

_N = 4096
_S = 200
_V = 100000
_NC = 2
_NS = 16
_L = 16
_NW = _NC * _NS
_ROWS = _N // _NW
_BLOCKS = _ROWS // _L
_VQ = _V // 4
_NSPMEM_Q = 4

_mesh = plsc.VectorSubcoreMesh(core_axis_name="c", subcore_axis_name="s")


def _bow_logits_body(x_hbm, wq_hbm, out_hbm, table_sh_a, table_sh_b,
                     table_v, idx_v, out_v, sem_x, sem_st):
    table_sh = (table_sh_a, table_sh_b)
    sid = lax.axis_index("s")
    wid = sid * _NC + lax.axis_index("c")
    base_row = wid * _ROWS
    cp_x = pltpu.make_async_copy(
        x_hbm.at[pl.ds(base_row * _S, _ROWS * _S)], idx_v, sem_x)
    cp_x.start()

    @pl.when(sid == 0)
    def _():
        pltpu.sync_copy(wq_hbm.at[0], table_sh[0])

    plsc.subcore_barrier()
    for q in range(_NSPMEM_Q):
        if q + 1 < _NSPMEM_Q:
            cp_st = pltpu.make_async_copy(
                wq_hbm.at[q + 1], table_sh[(q + 1) % 2], sem_st)

            @pl.when(sid == 0)
            def _(cp_st=cp_st):
                cp_st.start()

        pltpu.sync_copy(table_sh[q % 2], table_v.at[pl.ds(q * _VQ, _VQ)])
        if q + 1 < _NSPMEM_Q:
            @pl.when(sid == 0)
            def _(cp_st=cp_st):
                cp_st.wait()

            plsc.subcore_barrier()

    cp_x.wait()

    lane = lax.iota(jnp.int32, _L)
    row_offs = [(b * _L + lane) * _S for b in range(_BLOCKS)]
    zero = jnp.zeros((_L,), jnp.float32)

    @plsc.parallel_loop(0, _S, unroll=4, carry=(zero,) * _BLOCKS)
    def accs(s, acc):
        new = []
        for b in range(_BLOCKS):
            idx16 = plsc.load_gather(idx_v, [row_offs[b] + s])
            new.append(acc[b] + plsc.load_gather(table_v, [idx16]))
        return tuple(new)

    for b in range(_BLOCKS):
        out_v[pl.ds(b * _L, _L)] = accs[b]
    pltpu.sync_copy(out_v, out_hbm.at[pl.ds(base_row, _ROWS)])


_bow_logits = functools.partial(
    pl.kernel,
    mesh=_mesh,
    out_type=jax.ShapeDtypeStruct((_N,), jnp.float32),
    scratch_types=[
        pltpu.VMEM_SHARED((_VQ,), jnp.float32),
        pltpu.VMEM_SHARED((_VQ,), jnp.float32),
        pltpu.VMEM((_V,), jnp.float32),
        pltpu.VMEM((_ROWS * _S,), jnp.int32),
        pltpu.VMEM((_ROWS,), jnp.float32),
        pltpu.SemaphoreType.DMA,
        pltpu.SemaphoreType.DMA,
    ],
    compiler_params=pltpu.CompilerParams(needs_layout_passes=False),
)(_bow_logits_body)


def kernel(x, weight):
    xf = x.reshape(-1)
    wq = weight.reshape(4, _VQ)
    return _bow_logits(xf, wq)


# speedup vs baseline: 1.0003x; 1.0003x over previous
import functools

import jax
import jax.numpy as jnp
from jax import lax
from jax.experimental import pallas as pl
from jax.experimental.pallas import tpu as pltpu
from jax.experimental.pallas import tpu_sc as plsc

_N = 4096
_S = 200
_V = 100000
_NC = 2
_NS = 16
_L = 16
_NW = _NC * _NS
_ROWS = _N // _NW
_BLOCKS = _ROWS // _L
_VQ = _V // 4
_VQS = 12800
_NSPMEM_Q = 4

_mesh = plsc.VectorSubcoreMesh(core_axis_name="c", subcore_axis_name="s")


def _bow_logits_body(x_hbm, wq_hbm, out_hbm, table_sh_a, table_sh_b,
                     table_v, idx_v, out_v, sem_x, sem_st, sem_f):
    table_sh = (table_sh_a, table_sh_b)
    sid = lax.axis_index("s")
    wid = sid * _NC + lax.axis_index("c")
    base_row = wid * _ROWS
    cp_x = pltpu.make_async_copy(
        x_hbm.at[pl.ds(base_row * _S, _ROWS * _S)], idx_v, sem_x)
    cp_x.start()

    @pl.when(sid == 0)
    def _():
        pltpu.sync_copy(wq_hbm.at[0], table_sh[0])

    plsc.subcore_barrier()
    for q in range(_NSPMEM_Q):
        if q + 1 < _NSPMEM_Q:
            cp_st = pltpu.make_async_copy(
                wq_hbm.at[q + 1], table_sh[(q + 1) % 2], sem_st)

            @pl.when(sid == 0)
            def _(cp_st=cp_st):
                cp_st.start()

        cp_f0 = pltpu.make_async_copy(
            table_sh[q % 2].at[pl.ds(0, _VQS)],
            table_v.at[pl.ds(q * _VQ, _VQS)], sem_f)
        cp_f1 = pltpu.make_async_copy(
            table_sh[q % 2].at[pl.ds(_VQS, _VQ - _VQS)],
            table_v.at[pl.ds(q * _VQ + _VQS, _VQ - _VQS)], sem_f)
        cp_f0.start()
        cp_f1.start()
        cp_f0.wait()
        cp_f1.wait()
        if q + 1 < _NSPMEM_Q:
            @pl.when(sid == 0)
            def _(cp_st=cp_st):
                cp_st.wait()

            plsc.subcore_barrier()

    cp_x.wait()

    lane = lax.iota(jnp.int32, _L)
    row_offs = [(b * _L + lane) * _S for b in range(_BLOCKS)]
    zero = jnp.zeros((_L,), jnp.float32)

    @plsc.parallel_loop(0, _S, unroll=4, carry=(zero,) * _BLOCKS)
    def accs(s, acc):
        new = []
        for b in range(_BLOCKS):
            idx16 = plsc.load_gather(idx_v, [row_offs[b] + s])
            new.append(acc[b] + plsc.load_gather(table_v, [idx16]))
        return tuple(new)

    for b in range(_BLOCKS):
        out_v[pl.ds(b * _L, _L)] = accs[b]
    pltpu.sync_copy(out_v, out_hbm.at[pl.ds(base_row, _ROWS)])


_bow_logits = functools.partial(
    pl.kernel,
    mesh=_mesh,
    out_type=jax.ShapeDtypeStruct((_N,), jnp.float32),
    scratch_types=[
        pltpu.VMEM_SHARED((_VQ,), jnp.float32),
        pltpu.VMEM_SHARED((_VQ,), jnp.float32),
        pltpu.VMEM((_V,), jnp.float32),
        pltpu.VMEM((_ROWS * _S,), jnp.int32),
        pltpu.VMEM((_ROWS,), jnp.float32),
        pltpu.SemaphoreType.DMA,
        pltpu.SemaphoreType.DMA,
        pltpu.SemaphoreType.DMA,
    ],
    compiler_params=pltpu.CompilerParams(needs_layout_passes=False),
)(_bow_logits_body)


def kernel(x, weight):
    xf = x.reshape(-1)
    wq = weight.reshape(4, _VQ)
    return _bow_logits(xf, wq)


# speedup vs baseline: 1.0032x; 1.0028x over previous
import functools

import jax
import jax.numpy as jnp
from jax import lax
from jax.experimental import pallas as pl
from jax.experimental.pallas import tpu as pltpu
from jax.experimental.pallas import tpu_sc as plsc

_N = 4096
_S = 200
_V = 100000
_NC = 2
_NS = 16
_L = 16
_NW = _NC * _NS
_ROWS = _N // _NW
_BLOCKS = _ROWS // _L
_VQ = _V // 4
_NQ = 4

_mesh = plsc.VectorSubcoreMesh(core_axis_name="c", subcore_axis_name="s")


def _bow_logits_body(x_hbm, wq_hbm, out_hbm, table_sh_a, table_sh_b,
                     table_v, idx_v, out_v, sem_x, sem_st):
    table_sh = (table_sh_a, table_sh_b)
    sid = lax.axis_index("s")
    wid = sid * _NC + lax.axis_index("c")
    base_row = wid * _ROWS
    cp_x = pltpu.make_async_copy(
        x_hbm.at[pl.ds(base_row * _S, _ROWS * _S)], idx_v, sem_x)
    cp_x.start()

    @pl.when(sid == 0)
    def _():
        pltpu.sync_copy(wq_hbm.at[0], table_sh[0])

    plsc.subcore_barrier()
    for q in range(_NQ):
        if q + 1 < _NQ:
            cp_st = pltpu.make_async_copy(
                wq_hbm.at[q + 1], table_sh[(q + 1) % 2], sem_st)

            @pl.when(sid == 0)
            def _(cp_st=cp_st):
                cp_st.start()

        pltpu.sync_copy(table_sh[q % 2], table_v.at[pl.ds(q * _VQ, _VQ)])
        if q + 1 < _NQ:
            @pl.when(sid == 0)
            def _(cp_st=cp_st):
                cp_st.wait()

            plsc.subcore_barrier()

    cp_x.wait()

    lane = lax.iota(jnp.int32, _L)
    row_offs = [(b * _L + lane) * _S for b in range(_BLOCKS)]
    zero = jnp.zeros((_L,), jnp.float32)

    @plsc.parallel_loop(0, _S, unroll=2, carry=(zero,) * _BLOCKS)
    def accs(s, acc):
        new = []
        for b in range(_BLOCKS):
            idx16 = plsc.load_gather(idx_v, [row_offs[b] + s])
            new.append(acc[b] + plsc.load_gather(table_v, [idx16]))
        return tuple(new)

    for b in range(_BLOCKS):
        out_v[pl.ds(b * _L, _L)] = accs[b]
    pltpu.sync_copy(out_v, out_hbm.at[pl.ds(base_row, _ROWS)])


_bow_logits = functools.partial(
    pl.kernel,
    mesh=_mesh,
    out_type=jax.ShapeDtypeStruct((_N,), jnp.float32),
    scratch_types=[
        pltpu.VMEM_SHARED((_VQ,), jnp.float32),
        pltpu.VMEM_SHARED((_VQ,), jnp.float32),
        pltpu.VMEM((_V,), jnp.float32),
        pltpu.VMEM((_ROWS * _S,), jnp.int32),
        pltpu.VMEM((_ROWS,), jnp.float32),
        pltpu.SemaphoreType.DMA,
        pltpu.SemaphoreType.DMA,
    ],
    compiler_params=pltpu.CompilerParams(needs_layout_passes=False),
)(_bow_logits_body)


def kernel(x, weight):
    xf = x.reshape(-1)
    wq = weight.reshape(_NQ, _VQ)
    return _bow_logits(xf, wq)
